# baseline (speedup 1.0000x reference)
import functools

import jax
import jax.numpy as jnp
from jax import lax
from jax.experimental import pallas as pl
from jax.experimental.pallas import tpu as pltpu

HID = 256
N_NODES = 10000
NP = 10240
BR = 256
F32 = jnp.float32


def _leaky(x):
    return jnp.where(x >= 0, x, 0.01 * x)


def _dot(x, w):
    return jax.lax.dot_general(x, w, (((1,), (0,)), ((), ())),
                               preferred_element_type=F32)


def _mlp1_body(v_ref, w1, b1, w2, b2, w3, b3, w4, b4, feat_ref):
    x = v_ref[...]
    h = _leaky(_dot(x, w1[...]) + b1[...])
    h = _leaky(_dot(h, w2[...]) + b2[...])
    h = _leaky(_dot(h, w3[...]) + b3[...])
    h = _leaky(_dot(h, w4[...]) + b4[...])
    feat_ref[:, :HID] = h
    feat_ref[:, HID:] = x


def _mlp1_call(verts_p, ws):
    n = verts_p.shape[0]
    grid = (n // BR,)
    row = lambda i: (i, 0)
    zero = lambda i: (0, 0)
    in_specs = [pl.BlockSpec((BR, 128), row)]
    for (w, b) in ws:
        in_specs.append(pl.BlockSpec(w.shape, zero))
        in_specs.append(pl.BlockSpec(b.shape, zero))
    args = [verts_p]
    for (w, b) in ws:
        args += [w, b]
    return pl.pallas_call(
        _mlp1_body,
        grid=grid,
        in_specs=in_specs,
        out_specs=pl.BlockSpec((BR, HID + 128), row),
        out_shape=jax.ShapeDtypeStruct((n, HID + 128), F32),
    )(*args)


def _gconv_body(first, fin_ref, agg_ref, w0, b0, w1, b1, v0_ref, v1s_ref):
    if first:
        feat = fin_ref[...]
    else:
        agg = jnp.concatenate([agg_ref[0], agg_ref[1]], axis=-1)
        feat = jax.nn.relu(fin_ref[...] + agg)
    v0 = _dot(feat, w0[...]) + b0[...]
    v1 = _dot(feat, w1[...]) + b1[...]
    v0_ref[...] = v0
    v1s_ref[0] = v1[:, :128]
    v1s_ref[1] = v1[:, 128:]


def _gconv_call(fin, agg_s, w0, b0, w1, b1):
    n = fin.shape[0]
    first = agg_s is None
    grid = (n // BR,)
    row = lambda i: (i, 0)
    row3 = lambda i: (0, i, 0)
    zero = lambda i: (0, 0)
    din = fin.shape[1]
    in_specs = [pl.BlockSpec((BR, din), row),
                pl.BlockSpec((2, BR, 128), row3),
                pl.BlockSpec(w0.shape, zero), pl.BlockSpec(b0.shape, zero),
                pl.BlockSpec(w1.shape, zero), pl.BlockSpec(b1.shape, zero)]
    if first:
        agg_s = jnp.zeros((2, BR, 128), F32)
        in_specs[1] = pl.BlockSpec((2, BR, 128), lambda i: (0, 0, 0))
    return pl.pallas_call(
        functools.partial(_gconv_body, first),
        grid=grid,
        in_specs=in_specs,
        out_specs=[pl.BlockSpec((BR, HID), row),
                   pl.BlockSpec((2, BR, 128), row3)],
        out_shape=[jax.ShapeDtypeStruct((n, HID), F32),
                   jax.ShapeDtypeStruct((2, n, 128), F32)],
    )(fin, agg_s, w0, b0, w1, b1)


def _mlp3_body(v0_ref, agg_ref, w1, b1, w2, b2, w3, b3, w4, b4, out_ref):
    agg = jnp.concatenate([agg_ref[0], agg_ref[1]], axis=-1)
    h = jax.nn.relu(v0_ref[...] + agg)
    h = _leaky(_dot(h, w1[...]) + b1[...])
    h = _leaky(_dot(h, w2[...]) + b2[...])
    h = _leaky(_dot(h, w3[...]) + b3[...])
    h = _dot(h, w4[...]) + b4[...]
    out_ref[...] = h


def _mlp3_call(v0, agg_s, ws):
    n = v0.shape[0]
    grid = (n // BR,)
    row = lambda i: (i, 0)
    row3 = lambda i: (0, i, 0)
    zero = lambda i: (0, 0)
    in_specs = [pl.BlockSpec((BR, HID), row),
                pl.BlockSpec((2, BR, 128), row3)]
    args = [v0, agg_s]
    for (w, b) in ws:
        in_specs.append(pl.BlockSpec(w.shape, zero))
        in_specs.append(pl.BlockSpec(b.shape, zero))
        args += [w, b]
    return pl.pallas_call(
        _mlp3_body,
        grid=grid,
        in_specs=in_specs,
        out_specs=pl.BlockSpec((BR, 128), row),
        out_shape=jax.ShapeDtypeStruct((n, 128), F32),
    )(*args)


def _pad2(w, r, c):
    return jnp.pad(w, ((0, r - w.shape[0]), (0, c - w.shape[1])))


def _prep(params):
    mlp1 = []
    for (w, b) in params["mlp1"]:
        mlp1.append((_pad2(w, max(128, w.shape[0]), w.shape[1]),
                     b[None, :]))
    gconv = []
    for i, (w0, b0, w1, b1) in enumerate(params["gconv"]):
        if i == 0:
            w0 = _pad2(w0, 384, HID)
            w1 = _pad2(w1, 384, HID)
        gconv.append((w0, b0[None, :], w1, b1[None, :]))
    mlp3 = []
    for (w, b) in params["mlp3"]:
        mlp3.append((_pad2(w, max(128, w.shape[0]), 128),
                     _pad2(b[None, :], 1, 128)))
    return mlp1, gconv, mlp3


def kernel(verts, edges, params):
    mlp1, gconv, mlp3 = _prep(params)
    v3 = verts.reshape(-1, 3)
    verts_p = jnp.pad(v3, ((0, NP - N_NODES), (0, 128 - 3)))

    featp = _mlp1_call(verts_p, mlp1)

    src = edges[:, 0]
    dst = edges[:, 1]
    gidx = jnp.concatenate([dst, src])
    sidx = jnp.concatenate([src, dst])

    fin = featp
    agg_s = None
    for (w0, b0, w1, b1) in gconv:
        v0, v1s = _gconv_call(fin, agg_s, w0, b0, w1, b1)
        v1 = jnp.concatenate([v1s[0], v1s[1]], axis=1)
        agg = jnp.zeros((NP, HID), F32).at[sidx].add(v1[gidx])
        agg_s = jnp.stack([agg[:, :128], agg[:, 128:]])
        fin = v0

    out = _mlp3_call(fin, agg_s, mlp3)
    return out[:N_NODES, :3]


# speedup vs baseline: 7.6981x; 7.6981x over previous
import functools

import jax
import jax.numpy as jnp
from jax import lax
from jax.experimental import pallas as pl
from jax.experimental.pallas import tpu as pltpu
from jax.experimental.pallas import tpu_sc as plsc

HID = 256
N_NODES = 10000
NP = 10240
BR = 256
F32 = jnp.float32

NSUB = 16
NE2P = 327680
EPW = NE2P // NSUB
NCH = EPW // 128
RPW = NP // NSUB


def _leaky(x):
    return jnp.where(x >= 0, x, 0.01 * x)


def _dot(x, w):
    return jax.lax.dot_general(x, w, (((1,), (0,)), ((), ())),
                               preferred_element_type=F32)


def _mlp1_body(v_ref, w1, b1, w2, b2, w3, b3, w4, b4, feat_ref):
    x = v_ref[...]
    h = _leaky(_dot(x, w1[...]) + b1[...])
    h = _leaky(_dot(h, w2[...]) + b2[...])
    h = _leaky(_dot(h, w3[...]) + b3[...])
    h = _leaky(_dot(h, w4[...]) + b4[...])
    feat_ref[:, :HID] = h
    feat_ref[:, HID:] = x


def _mlp1_call(verts_p, ws):
    n = verts_p.shape[0]
    grid = (n // BR,)
    row = lambda i: (i, 0)
    zero = lambda i: (0, 0)
    in_specs = [pl.BlockSpec((BR, 128), row)]
    for (w, b) in ws:
        in_specs.append(pl.BlockSpec(w.shape, zero))
        in_specs.append(pl.BlockSpec(b.shape, zero))
    args = [verts_p]
    for (w, b) in ws:
        args += [w, b]
    return pl.pallas_call(
        _mlp1_body,
        grid=grid,
        in_specs=in_specs,
        out_specs=pl.BlockSpec((BR, HID + 128), row),
        out_shape=jax.ShapeDtypeStruct((n, HID + 128), F32),
    )(*args)


def _gconv_body(first, fin_ref, agg_ref, w0, b0, w1, b1, v0_ref, v1s_ref):
    if first:
        feat = fin_ref[...]
    else:
        agg = jnp.concatenate([agg_ref[0], agg_ref[1]], axis=-1)
        feat = jax.nn.relu(fin_ref[...] + agg)
    v0 = _dot(feat, w0[...]) + b0[...]
    v1 = _dot(feat, w1[...]) + b1[...]
    v0_ref[...] = v0
    v1s_ref[0] = v1[:, :128]
    v1s_ref[1] = v1[:, 128:]


def _gconv_call(fin, agg_s, w0, b0, w1, b1):
    n = fin.shape[0]
    first = agg_s is None
    grid = (n // BR,)
    row = lambda i: (i, 0)
    row3 = lambda i: (0, i, 0)
    zero = lambda i: (0, 0)
    din = fin.shape[1]
    in_specs = [pl.BlockSpec((BR, din), row),
                pl.BlockSpec((2, BR, 128), row3),
                pl.BlockSpec(w0.shape, zero), pl.BlockSpec(b0.shape, zero),
                pl.BlockSpec(w1.shape, zero), pl.BlockSpec(b1.shape, zero)]
    if first:
        agg_s = jnp.zeros((2, BR, 128), F32)
        in_specs[1] = pl.BlockSpec((2, BR, 128), lambda i: (0, 0, 0))
    return pl.pallas_call(
        functools.partial(_gconv_body, first),
        grid=grid,
        in_specs=in_specs,
        out_specs=[pl.BlockSpec((BR, HID), row),
                   pl.BlockSpec((2, BR, 128), row3)],
        out_shape=[jax.ShapeDtypeStruct((n, HID), F32),
                   jax.ShapeDtypeStruct((2, n, 128), F32)],
    )(fin, agg_s, w0, b0, w1, b1)


def _mlp3_body(v0_ref, agg_ref, w1, b1, w2, b2, w3, b3, w4, b4, out_ref):
    agg = jnp.concatenate([agg_ref[0], agg_ref[1]], axis=-1)
    h = jax.nn.relu(v0_ref[...] + agg)
    h = _leaky(_dot(h, w1[...]) + b1[...])
    h = _leaky(_dot(h, w2[...]) + b2[...])
    h = _leaky(_dot(h, w3[...]) + b3[...])
    h = _dot(h, w4[...]) + b4[...]
    out_ref[...] = h


def _mlp3_call(v0, agg_s, ws):
    n = v0.shape[0]
    grid = (n // BR,)
    row = lambda i: (i, 0)
    row3 = lambda i: (0, i, 0)
    zero = lambda i: (0, 0)
    in_specs = [pl.BlockSpec((BR, HID), row),
                pl.BlockSpec((2, BR, 128), row3)]
    args = [v0, agg_s]
    for (w, b) in ws:
        in_specs.append(pl.BlockSpec(w.shape, zero))
        in_specs.append(pl.BlockSpec(b.shape, zero))
        args += [w, b]
    return pl.pallas_call(
        _mlp3_body,
        grid=grid,
        in_specs=in_specs,
        out_specs=pl.BlockSpec((BR, 128), row),
        out_shape=jax.ShapeDtypeStruct((n, 128), F32),
    )(*args)


NCHB = 32
NBLK = NCH // NCHB


def _agg_body(v1_hbm, g_hbm, s_hbm, z_hbm, out_hbm,
              gidx_v, sidx_v, rows_a, rows_b, acc_sh, sem_a, sem_b):
    c = lax.axis_index("c")
    s = lax.axis_index("s")
    pltpu.sync_copy(z_hbm, acc_sh.at[pl.ds(s * RPW, RPW)])
    plsc.subcore_barrier()

    def blk(b, _):
        pltpu.sync_copy(g_hbm.at[c, s, pl.ds(b * NCHB, NCHB)], gidx_v)
        pltpu.sync_copy(s_hbm.at[s, pl.ds(b * NCHB, NCHB)], sidx_v)
        pltpu.async_copy(v1_hbm.at[gidx_v.at[0]], rows_a, sem_a)

        def pair(i, _):
            j = 2 * i
            pltpu.async_copy(v1_hbm.at[gidx_v.at[j + 1]], rows_b, sem_b)
            pltpu.make_async_copy(v1_hbm.at[gidx_v.at[j]], rows_a,
                                  sem_a).wait()
            pltpu.sync_copy(rows_a, acc_sh.at[sidx_v.at[j]], add=True)
            pltpu.async_copy(v1_hbm.at[gidx_v.at[(j + 2) % NCHB]],
                             rows_a, sem_a)
            pltpu.make_async_copy(v1_hbm.at[gidx_v.at[j + 1]], rows_b,
                                  sem_b).wait()
            pltpu.sync_copy(rows_b, acc_sh.at[sidx_v.at[j + 1]], add=True)
            return 0

        lax.fori_loop(0, NCHB // 2, pair, 0)
        pltpu.make_async_copy(v1_hbm.at[gidx_v.at[0]], rows_a, sem_a).wait()
        return 0

    lax.fori_loop(0, NBLK, blk, 0)
    plsc.subcore_barrier()
    pltpu.sync_copy(acc_sh.at[pl.ds(s * RPW, RPW)],
                    out_hbm.at[c, pl.ds(s * RPW, RPW)])


_AGG_CALL = pl.kernel(
    _agg_body,
    out_type=jax.ShapeDtypeStruct((2, NP, 128), F32),
    mesh=plsc.VectorSubcoreMesh(core_axis_name="c", subcore_axis_name="s"),
    scratch_types=[
        pltpu.VMEM((NCHB, 128), jnp.int32),
        pltpu.VMEM((NCHB, 128), jnp.int32),
        pltpu.VMEM((128, 128), F32),
        pltpu.VMEM((128, 128), F32),
        pltpu.VMEM_SHARED((NP, 128), F32),
        pltpu.SemaphoreType.DMA,
        pltpu.SemaphoreType.DMA,
    ],
)


def _edge_indices(edges):
    src = edges[:, 0]
    dst = edges[:, 1]
    gidx = jnp.concatenate([dst, src])
    sidx = jnp.concatenate([src, dst])
    npad = NE2P - gidx.shape[0]
    padi = N_NODES + (jnp.arange(npad, dtype=jnp.int32) % (NP - N_NODES))
    gidx = jnp.concatenate([gidx, padi])
    sidx = jnp.concatenate([sidx, padi])
    g2 = jnp.stack([gidx, gidx + NP]).reshape(2, NSUB, NCH, 128)
    s2 = sidx.reshape(NSUB, NCH, 128)
    return g2, s2


def _pad2(w, r, c):
    return jnp.pad(w, ((0, r - w.shape[0]), (0, c - w.shape[1])))


def _prep(params):
    mlp1 = []
    for (w, b) in params["mlp1"]:
        mlp1.append((_pad2(w, max(128, w.shape[0]), w.shape[1]),
                     b[None, :]))
    gconv = []
    for i, (w0, b0, w1, b1) in enumerate(params["gconv"]):
        if i == 0:
            w0 = _pad2(w0, 384, HID)
            w1 = _pad2(w1, 384, HID)
        gconv.append((w0, b0[None, :], w1, b1[None, :]))
    mlp3 = []
    for (w, b) in params["mlp3"]:
        mlp3.append((_pad2(w, max(128, w.shape[0]), 128),
                     _pad2(b[None, :], 1, 128)))
    return mlp1, gconv, mlp3


def kernel(verts, edges, params):
    mlp1, gconv, mlp3 = _prep(params)
    v3 = verts.reshape(-1, 3)
    verts_p = jnp.pad(v3, ((0, NP - N_NODES), (0, 128 - 3)))

    featp = _mlp1_call(verts_p, mlp1)

    g2, s2 = _edge_indices(edges)
    zeros_blk = jnp.zeros((RPW, 128), F32)

    fin = featp
    agg_s = None
    for (w0, b0, w1, b1) in gconv:
        v0, v1s = _gconv_call(fin, agg_s, w0, b0, w1, b1)
        agg_s = _AGG_CALL(v1s.reshape(2 * NP, 128), g2, s2, zeros_blk)
        fin = v0

    out = _mlp3_call(fin, agg_s, mlp3)
    return out[:N_NODES, :3]
